# baseline (speedup 1.0000x reference)
import functools

import jax
import jax.numpy as jnp
from jax import lax
from jax.experimental import pallas as pl
from jax.experimental.pallas import tpu as pltpu
from jax.experimental.pallas import tpu_sc as plsc

E = 8
K = 2
T = 4096
D = 1024
H = 4096
N = T * K
B = 256
LOGB = 8
SLOTS = N + E * B
G = SLOTS // B
JMAX = N // B + 1
L = 16
NTILES = 32


def _router_body(x_ref, rw_ref, rb_ref, idx_ref, gate_ref, cnt_ref, xlin_ref):
    t = pl.program_id(0)
    xb = x_ref[...]
    xlin_ref[...] = xb
    logits = (
        jax.lax.dot_general(
            xb, rw_ref[...], (((1,), (1,)), ((), ())),
            preferred_element_type=jnp.float32,
        )
        + rb_ref[...][None, :]
    )
    m1 = jnp.max(logits, axis=-1)
    a1 = jnp.argmax(logits, axis=-1).astype(jnp.int32)
    cols = jax.lax.broadcasted_iota(jnp.int32, logits.shape, 1)
    masked = jnp.where(cols == a1[:, None], -jnp.inf, logits)
    m2 = jnp.max(masked, axis=-1)
    a2 = jnp.argmax(masked, axis=-1).astype(jnp.int32)
    e2 = jnp.exp(m2 - m1)
    denom = 1.0 + e2
    idx_ref[...] = jnp.stack([a1, a2], axis=-1)
    gate_ref[...] = jnp.stack([1.0 / denom, e2 / denom], axis=-1)

    cols16 = jax.lax.broadcasted_iota(jnp.int32, (xb.shape[0], L), 1)
    hist = jnp.sum(
        (cols16 == a1[:, None]).astype(jnp.int32)
        + (cols16 == a2[:, None]).astype(jnp.int32),
        axis=0,
    )[None, :]

    @pl.when(t == 0)
    def _():
        cnt_ref[...] = hist

    @pl.when(t > 0)
    def _():
        cnt_ref[...] += hist


def _router(x, router_w, router_b):
    return pl.pallas_call(
        _router_body,
        grid=(4,),
        in_specs=[
            pl.BlockSpec((T // 4, D), lambda t: (t, 0)),
            pl.BlockSpec((E, D), lambda t: (0, 0)),
            pl.BlockSpec((E,), lambda t: (0,)),
        ],
        out_specs=[
            pl.BlockSpec((T // 4, K), lambda t: (t, 0)),
            pl.BlockSpec((T // 4, K), lambda t: (t, 0)),
            pl.BlockSpec((1, L), lambda t: (0, 0)),
            pl.BlockSpec((T // 4, D), lambda t: (t, 0)),
        ],
        out_shape=[
            jax.ShapeDtypeStruct((T, K), jnp.int32),
            jax.ShapeDtypeStruct((T, K), jnp.float32),
            jax.ShapeDtypeStruct((1, L), jnp.int32),
            jax.ShapeDtypeStruct((T, D), jnp.float32),
        ],
    )(x, router_w, router_b)


def _dispatch_body(ea_hbm, gf_hbm, cnt_hbm, pos0_hbm, pos1_hbm, tos_hbm,
                   sg_hbm, nblk_hbm, poffb_hbm,
                   ea_v, gf_v, pos_v, pos0_v, pos1_v, tos_v, sg_v, rp_v,
                   misc_v):
    wid = lax.axis_index("s") * 2 + lax.axis_index("c")

    @pl.when(wid == 0)
    def _():
        pltpu.sync_copy(ea_hbm, ea_v)
        pltpu.sync_copy(gf_hbm, gf_v)
        pltpu.sync_copy(cnt_hbm.at[0], misc_v.at[0])

        lanes = lax.iota(jnp.int32, L)
        counts = misc_v[0, :]
        padded = ((counts + (B - 1)) >> LOGB) << LOGB
        poff = plsc.cumsum(padded) - padded
        misc_v[1, :] = padded >> LOGB
        misc_v[2, :] = poff >> LOGB
        rp_v[...] = poff

        def zero_body(i, _):
            tos_v[pl.ds(i * L, L)] = jnp.zeros((L,), jnp.int32)
            return 0
        lax.fori_loop(0, SLOTS // L, zero_body, 0)

        def zero_sg(i, _):
            sg_v[pl.ds(i * L, L)] = jnp.zeros((L,), jnp.float32)
            return 0
        lax.fori_loop(0, (SLOTS + B) // L, zero_sg, 0)

        def chunk_body(c, _):
            v = ea_v[pl.ds(c * L, L)]
            rk = jnp.zeros((L,), jnp.int32)
            rp = rp_v[...]
            for e in range(E):
                m = v == e
                ones = jnp.where(m, 1, 0)
                cs = plsc.cumsum(ones)
                rk = jnp.where(m, cs - 1, rk)
                tot = jnp.sum(ones)
                rp = jnp.where(lanes == e, rp + tot, rp)
            base = plsc.load_gather(rp_v, [v])
            rp_v[...] = rp
            posc = base + rk
            pos_v[pl.ds(c * L, L)] = posc
            tok = (c * (L // K)) + (lanes >> 1)
            plsc.store_scatter(tos_v, [posc], tok)
            gc = gf_v[pl.ds(c * L, L)]
            plsc.store_scatter(sg_v, [posc], gc)
            return 0
        lax.fori_loop(0, N // L, chunk_body, 0)

        def split_body(c, _):
            ti = c * L + lanes
            pos0_v[pl.ds(c * L, L)] = plsc.load_gather(pos_v, [ti * 2])
            pos1_v[pl.ds(c * L, L)] = plsc.load_gather(pos_v, [ti * 2 + 1])
            return 0
        lax.fori_loop(0, T // L, split_body, 0)

        pltpu.sync_copy(pos0_v, pos0_hbm)
        pltpu.sync_copy(pos1_v, pos1_hbm)
        pltpu.sync_copy(tos_v, tos_hbm)
        pltpu.sync_copy(sg_v, sg_hbm)
        pltpu.sync_copy(misc_v.at[1], nblk_hbm)
        pltpu.sync_copy(misc_v.at[2], poffb_hbm)


def _dispatch(ea, gates_flat, counts):
    fn = pl.kernel(
        _dispatch_body,
        out_type=[
            jax.ShapeDtypeStruct((T,), jnp.int32),
            jax.ShapeDtypeStruct((T,), jnp.int32),
            jax.ShapeDtypeStruct((SLOTS,), jnp.int32),
            jax.ShapeDtypeStruct((SLOTS + B,), jnp.float32),
            jax.ShapeDtypeStruct((L,), jnp.int32),
            jax.ShapeDtypeStruct((L,), jnp.int32),
        ],
        mesh=plsc.VectorSubcoreMesh(core_axis_name="c", subcore_axis_name="s"),
        scratch_types=[
            pltpu.VMEM((N,), jnp.int32),
            pltpu.VMEM((N,), jnp.float32),
            pltpu.VMEM((N,), jnp.int32),
            pltpu.VMEM((T,), jnp.int32),
            pltpu.VMEM((T,), jnp.int32),
            pltpu.VMEM((SLOTS,), jnp.int32),
            pltpu.VMEM((SLOTS + B,), jnp.float32),
            pltpu.VMEM((L,), jnp.int32),
            pltpu.VMEM((3, L), jnp.int32),
        ],
        compiler_params=pltpu.CompilerParams(needs_layout_passes=False),
    )
    return fn(ea, gates_flat, counts)


CHUNK = 16
NBUF = 6
PER_TILE = SLOTS // NTILES
NSTEP = PER_TILE // CHUNK


def _gather_body(x_hbm, tos_hbm, xs_hbm, idx_v, *bufs_and_sems):
    rows = bufs_and_sems[:NBUF]
    gsem = bufs_and_sems[NBUF:2 * NBUF]
    osem = bufs_and_sems[2 * NBUF:3 * NBUF]
    wid = lax.axis_index("s") * 2 + lax.axis_index("c")
    base = wid * PER_TILE
    pltpu.sync_copy(tos_hbm.at[pl.ds(base, PER_TILE)], idx_v)

    for p in range(NBUF):
        pltpu.async_copy(
            x_hbm.at[idx_v.at[pl.ds(p * CHUNK, CHUNK)]], rows[p], gsem[p])

    for i in range(NSTEP):
        p = i % NBUF
        pltpu.make_async_copy(
            x_hbm.at[idx_v.at[pl.ds(0, CHUNK)]], rows[p], gsem[p]).wait()
        pltpu.async_copy(
            rows[p], xs_hbm.at[pl.ds(base + i * CHUNK, CHUNK)], osem[p])
        if i + NBUF < NSTEP:
            pltpu.make_async_copy(
                rows[p], xs_hbm.at[pl.ds(0, CHUNK)], osem[p]).wait()
            pltpu.async_copy(
                x_hbm.at[idx_v.at[pl.ds((i + NBUF) * CHUNK, CHUNK)]],
                rows[p], gsem[p])
    for p in range(NBUF):
        pltpu.make_async_copy(
            rows[p], xs_hbm.at[pl.ds(0, CHUNK)], osem[p]).wait()


def _gather_xs(x, tos):
    fn = pl.kernel(
        _gather_body,
        out_type=jax.ShapeDtypeStruct((SLOTS, D), jnp.float32),
        mesh=plsc.VectorSubcoreMesh(core_axis_name="c", subcore_axis_name="s"),
        scratch_types=(
            [pltpu.VMEM((PER_TILE,), jnp.int32)]
            + [pltpu.VMEM((CHUNK, D), jnp.float32)] * NBUF
            + [pltpu.SemaphoreType.DMA] * (2 * NBUF)
        ),
    )
    return fn(x, tos)


def _ffn_body(nblk_ref, poffb_ref, xs_ref, sg_ref, w1_hbm, b1_ref, w2_hbm,
              b2_ref, o_ref, w1s, w2s, sem1, sem2):
    e = pl.program_id(0)
    j = pl.program_id(1)

    @pl.when((j == 0) & (nblk_ref[e] > 0))
    def _():
        pltpu.make_async_copy(w1_hbm.at[e], w1s, sem1).start()
        pltpu.make_async_copy(w2_hbm.at[e], w2s, sem2).start()

    @pl.when(j < nblk_ref[e])
    def _():
        xb = xs_ref[...]

        @pl.when(j == 0)
        def _():
            pltpu.make_async_copy(w1_hbm.at[e], w1s, sem1).wait()
        hid = jax.lax.dot_general(
            xb, w1s[...], (((1,), (1,)), ((), ())),
            preferred_element_type=jnp.float32,
        ) + b1_ref[0]
        hid = jnp.maximum(hid, 0.0)

        @pl.when(j == 0)
        def _():
            pltpu.make_async_copy(w2_hbm.at[e], w2s, sem2).wait()
        o_ref[...] = (
            jax.lax.dot_general(
                hid, w2s[...], (((1,), (1,)), ((), ())),
                preferred_element_type=jnp.float32,
            ) + b2_ref[0]
        ) * sg_ref[0, 0][:, None]


def _ffn(xs, sg, nblk, poffb, w1, b1, w2, b2):
    grid_spec = pltpu.PrefetchScalarGridSpec(
        num_scalar_prefetch=2,
        grid=(E, JMAX),
        in_specs=[
            pl.BlockSpec(
                (B, D),
                lambda e, j, nblk, poffb: (jnp.minimum(poffb[e] + j, G - 1), 0),
            ),
            pl.BlockSpec(
                (1, 1, B),
                lambda e, j, nblk, poffb: (
                    jnp.where(j < nblk[e], poffb[e] + j, G), 0, 0),
            ),
            pl.BlockSpec(memory_space=pl.ANY),
            pl.BlockSpec((1, 1, H), lambda e, j, nblk, poffb: (e, 0, 0)),
            pl.BlockSpec(memory_space=pl.ANY),
            pl.BlockSpec((1, 1, D), lambda e, j, nblk, poffb: (e, 0, 0)),
        ],
        out_specs=pl.BlockSpec(
            (B, D),
            lambda e, j, nblk, poffb: (
                jnp.where(j < nblk[e], poffb[e] + j, G), 0),
        ),
        scratch_shapes=[
            pltpu.VMEM((H, D), jnp.float32),
            pltpu.VMEM((D, H), jnp.float32),
            pltpu.SemaphoreType.DMA,
            pltpu.SemaphoreType.DMA,
        ],
    )
    y = pl.pallas_call(
        _ffn_body,
        grid_spec=grid_spec,
        out_shape=jax.ShapeDtypeStruct((SLOTS + B, D), jnp.float32),
    )(nblk, poffb, xs, sg.reshape(G + 1, 1, B), w1, b1.reshape(E, 1, H),
      w2, b2.reshape(E, 1, D))
    return y


TCHUNK = 16
TOK_PER_TILE = T // NTILES


CSTEP = TOK_PER_TILE // TCHUNK


def _combine_body(yg_hbm, pos0_hbm, pos1_hbm, out_hbm, i0_v, i1_v,
                  r0a_v, r0b_v, r1a_v, r1b_v,
                  g0a, g0b, g1a, g1b, oa, ob):
    wid = lax.axis_index("s") * 2 + lax.axis_index("c")
    base = wid * TOK_PER_TILE
    pltpu.sync_copy(pos0_hbm.at[pl.ds(base, TOK_PER_TILE)], i0_v)
    pltpu.sync_copy(pos1_hbm.at[pl.ds(base, TOK_PER_TILE)], i1_v)
    r0 = (r0a_v, r0b_v)
    r1 = (r1a_v, r1b_v)
    g0 = (g0a, g0b)
    g1 = (g1a, g1b)
    osem = (oa, ob)

    for p in range(2):
        pltpu.async_copy(
            yg_hbm.at[i0_v.at[pl.ds(p * TCHUNK, TCHUNK)]], r0[p], g0[p])
        pltpu.async_copy(
            yg_hbm.at[i1_v.at[pl.ds(p * TCHUNK, TCHUNK)]], r1[p], g1[p])

    for i in range(CSTEP):
        p = i % 2
        pltpu.make_async_copy(
            yg_hbm.at[i0_v.at[pl.ds(0, TCHUNK)]], r0[p], g0[p]).wait()
        pltpu.make_async_copy(
            yg_hbm.at[i1_v.at[pl.ds(0, TCHUNK)]], r1[p], g1[p]).wait()

        def add_row(r, _, p=p):
            for q in range(D // L):
                r0[p][r, pl.ds(q * L, L)] += r1[p][r, pl.ds(q * L, L)]
            return 0
        lax.fori_loop(0, TCHUNK, add_row, 0)
        pltpu.async_copy(
            r0[p], out_hbm.at[pl.ds(base + i * TCHUNK, TCHUNK)], osem[p])
        if i + 2 < CSTEP:
            pltpu.make_async_copy(
                r0[p], out_hbm.at[pl.ds(0, TCHUNK)], osem[p]).wait()
            pltpu.async_copy(
                yg_hbm.at[i0_v.at[pl.ds((i + 2) * TCHUNK, TCHUNK)]],
                r0[p], g0[p])
            pltpu.async_copy(
                yg_hbm.at[i1_v.at[pl.ds((i + 2) * TCHUNK, TCHUNK)]],
                r1[p], g1[p])
    for p in range(2):
        pltpu.make_async_copy(
            r0[p], out_hbm.at[pl.ds(0, TCHUNK)], osem[p]).wait()


def _combine(yg, pos0, pos1):
    fn = pl.kernel(
        _combine_body,
        out_type=jax.ShapeDtypeStruct((T, D), jnp.float32),
        mesh=plsc.VectorSubcoreMesh(core_axis_name="c", subcore_axis_name="s"),
        scratch_types=[
            pltpu.VMEM((TOK_PER_TILE,), jnp.int32),
            pltpu.VMEM((TOK_PER_TILE,), jnp.int32),
            pltpu.VMEM((TCHUNK, D), jnp.float32),
            pltpu.VMEM((TCHUNK, D), jnp.float32),
            pltpu.VMEM((TCHUNK, D), jnp.float32),
            pltpu.VMEM((TCHUNK, D), jnp.float32),
            pltpu.SemaphoreType.DMA,
            pltpu.SemaphoreType.DMA,
            pltpu.SemaphoreType.DMA,
            pltpu.SemaphoreType.DMA,
            pltpu.SemaphoreType.DMA,
            pltpu.SemaphoreType.DMA,
        ],
    )
    return fn(yg, pos0, pos1)


def kernel(x, router_w, router_b, noisy_w, noisy_b, w1, b1, w2, b2):
    del noisy_w, noisy_b
    topk_idx, gates, counts, xlin = _router(x, router_w, router_b)
    pos0, pos1, tos, sg, nblk, poffb = _dispatch(
        topk_idx.reshape(N), gates.reshape(N), counts)
    xs = _gather_xs(xlin, tos)
    yg = _ffn(xs, sg, nblk, poffb, w1, b1, w2, b2)
    return _combine(yg, pos0, pos1)


# speedup vs baseline: 1.0305x; 1.0305x over previous
import functools

import jax
import jax.numpy as jnp
from jax import lax
from jax.experimental import pallas as pl
from jax.experimental.pallas import tpu as pltpu
from jax.experimental.pallas import tpu_sc as plsc

E = 8
K = 2
T = 4096
D = 1024
H = 4096
N = T * K
B = 256
LOGB = 8
SLOTS = N + E * B
G = SLOTS // B
JMAX = N // B + 1
L = 16
NTILES = 32


def _router_body(x_ref, rw_ref, rb_ref, idx_ref, gate_ref, cnt_ref):
    t = pl.program_id(0)
    xb = x_ref[...]
    logits = (
        jax.lax.dot_general(
            xb, rw_ref[...], (((1,), (1,)), ((), ())),
            preferred_element_type=jnp.float32,
        )
        + rb_ref[...][None, :]
    )
    m1 = jnp.max(logits, axis=-1)
    a1 = jnp.argmax(logits, axis=-1).astype(jnp.int32)
    cols = jax.lax.broadcasted_iota(jnp.int32, logits.shape, 1)
    masked = jnp.where(cols == a1[:, None], -jnp.inf, logits)
    m2 = jnp.max(masked, axis=-1)
    a2 = jnp.argmax(masked, axis=-1).astype(jnp.int32)
    e2 = jnp.exp(m2 - m1)
    denom = 1.0 + e2
    idx_ref[...] = jnp.stack([a1, a2], axis=-1)
    gate_ref[...] = jnp.stack([1.0 / denom, e2 / denom], axis=-1)

    cols16 = jax.lax.broadcasted_iota(jnp.int32, (xb.shape[0], L), 1)
    hist = jnp.sum(
        (cols16 == a1[:, None]).astype(jnp.int32)
        + (cols16 == a2[:, None]).astype(jnp.int32),
        axis=0,
    )[None, :]

    @pl.when(t == 0)
    def _():
        cnt_ref[...] = hist

    @pl.when(t > 0)
    def _():
        cnt_ref[...] += hist


def _router(x, router_w, router_b):
    return pl.pallas_call(
        _router_body,
        grid=(4,),
        in_specs=[
            pl.BlockSpec((T // 4, D), lambda t: (t, 0)),
            pl.BlockSpec((E, D), lambda t: (0, 0)),
            pl.BlockSpec((E,), lambda t: (0,)),
        ],
        out_specs=[
            pl.BlockSpec((T // 4, K), lambda t: (t, 0)),
            pl.BlockSpec((T // 4, K), lambda t: (t, 0)),
            pl.BlockSpec((1, L), lambda t: (0, 0)),
        ],
        out_shape=[
            jax.ShapeDtypeStruct((T, K), jnp.int32),
            jax.ShapeDtypeStruct((T, K), jnp.float32),
            jax.ShapeDtypeStruct((1, L), jnp.int32),
        ],
    )(x, router_w, router_b)


def _dispatch_body(ea_hbm, gf_hbm, cnt_hbm, pos0_hbm, pos1_hbm, tos_hbm,
                   sg_hbm, nblk_hbm, poffb_hbm,
                   ea_v, gf_v, pos_v, pos0_v, pos1_v, tos_v, sg_v, rp_v,
                   misc_v):
    wid = lax.axis_index("s") * 2 + lax.axis_index("c")

    @pl.when(wid == 0)
    def _():
        pltpu.sync_copy(ea_hbm, ea_v)
        pltpu.sync_copy(gf_hbm, gf_v)
        pltpu.sync_copy(cnt_hbm.at[0], misc_v.at[0])

        lanes = lax.iota(jnp.int32, L)
        counts = misc_v[0, :]
        padded = ((counts + (B - 1)) >> LOGB) << LOGB
        poff = plsc.cumsum(padded) - padded
        misc_v[1, :] = padded >> LOGB
        misc_v[2, :] = poff >> LOGB
        rp_v[...] = poff

        def zero_body(i, _):
            tos_v[pl.ds(i * L, L)] = jnp.zeros((L,), jnp.int32)
            return 0
        lax.fori_loop(0, SLOTS // L, zero_body, 0)

        def zero_sg(i, _):
            sg_v[pl.ds(i * L, L)] = jnp.zeros((L,), jnp.float32)
            return 0
        lax.fori_loop(0, (SLOTS + B) // L, zero_sg, 0)

        def chunk_body(c, _):
            v = ea_v[pl.ds(c * L, L)]
            rk = jnp.zeros((L,), jnp.int32)
            rp = rp_v[...]
            for e in range(E):
                m = v == e
                ones = jnp.where(m, 1, 0)
                cs = plsc.cumsum(ones)
                rk = jnp.where(m, cs - 1, rk)
                tot = jnp.sum(ones)
                rp = jnp.where(lanes == e, rp + tot, rp)
            base = plsc.load_gather(rp_v, [v])
            rp_v[...] = rp
            posc = base + rk
            pos_v[pl.ds(c * L, L)] = posc
            tok = (c * (L // K)) + (lanes >> 1)
            plsc.store_scatter(tos_v, [posc], tok)
            gc = gf_v[pl.ds(c * L, L)]
            plsc.store_scatter(sg_v, [posc], gc)
            return 0
        lax.fori_loop(0, N // L, chunk_body, 0)

        def split_body(c, _):
            ti = c * L + lanes
            pos0_v[pl.ds(c * L, L)] = plsc.load_gather(pos_v, [ti * 2])
            pos1_v[pl.ds(c * L, L)] = plsc.load_gather(pos_v, [ti * 2 + 1])
            return 0
        lax.fori_loop(0, T // L, split_body, 0)

        pltpu.sync_copy(pos0_v, pos0_hbm)
        pltpu.sync_copy(pos1_v, pos1_hbm)
        pltpu.sync_copy(tos_v, tos_hbm)
        pltpu.sync_copy(sg_v, sg_hbm)
        pltpu.sync_copy(misc_v.at[1], nblk_hbm)
        pltpu.sync_copy(misc_v.at[2], poffb_hbm)


def _dispatch(ea, gates_flat, counts):
    fn = pl.kernel(
        _dispatch_body,
        out_type=[
            jax.ShapeDtypeStruct((T,), jnp.int32),
            jax.ShapeDtypeStruct((T,), jnp.int32),
            jax.ShapeDtypeStruct((SLOTS,), jnp.int32),
            jax.ShapeDtypeStruct((SLOTS + B,), jnp.float32),
            jax.ShapeDtypeStruct((L,), jnp.int32),
            jax.ShapeDtypeStruct((L,), jnp.int32),
        ],
        mesh=plsc.VectorSubcoreMesh(core_axis_name="c", subcore_axis_name="s"),
        scratch_types=[
            pltpu.VMEM((N,), jnp.int32),
            pltpu.VMEM((N,), jnp.float32),
            pltpu.VMEM((N,), jnp.int32),
            pltpu.VMEM((T,), jnp.int32),
            pltpu.VMEM((T,), jnp.int32),
            pltpu.VMEM((SLOTS,), jnp.int32),
            pltpu.VMEM((SLOTS + B,), jnp.float32),
            pltpu.VMEM((L,), jnp.int32),
            pltpu.VMEM((3, L), jnp.int32),
        ],
        compiler_params=pltpu.CompilerParams(needs_layout_passes=False),
    )
    return fn(ea, gates_flat, counts)


CHUNK = 16
NBUF = 6
PER_TILE = SLOTS // NTILES
NSTEP = PER_TILE // CHUNK


HALF = CHUNK // 2


def _fire_half_gathers(x_hbm, idx_v, rows, gsa, gsb, i, p):
    pltpu.async_copy(
        x_hbm.at[idx_v.at[pl.ds(i * CHUNK, HALF)]],
        rows[p].at[pl.ds(0, HALF)], gsa[p])
    pltpu.async_copy(
        x_hbm.at[idx_v.at[pl.ds(i * CHUNK + HALF, HALF)]],
        rows[p].at[pl.ds(HALF, HALF)], gsb[p])


def _gather_body(x_hbm, tos_hbm, xs_hbm, idx_v, *bufs_and_sems):
    rows = bufs_and_sems[:NBUF]
    gsa = bufs_and_sems[NBUF:2 * NBUF]
    gsb = bufs_and_sems[2 * NBUF:3 * NBUF]
    osem = bufs_and_sems[3 * NBUF:4 * NBUF]
    wid = lax.axis_index("s") * 2 + lax.axis_index("c")
    base = wid * PER_TILE
    pltpu.sync_copy(tos_hbm.at[pl.ds(base, PER_TILE)], idx_v)

    for p in range(NBUF):
        _fire_half_gathers(x_hbm, idx_v, rows, gsa, gsb, p, p)

    for i in range(NSTEP):
        p = i % NBUF
        pltpu.make_async_copy(
            x_hbm.at[idx_v.at[pl.ds(0, HALF)]],
            rows[p].at[pl.ds(0, HALF)], gsa[p]).wait()
        pltpu.make_async_copy(
            x_hbm.at[idx_v.at[pl.ds(0, HALF)]],
            rows[p].at[pl.ds(HALF, HALF)], gsb[p]).wait()
        pltpu.async_copy(
            rows[p], xs_hbm.at[pl.ds(base + i * CHUNK, CHUNK)], osem[p])
        if i + NBUF < NSTEP:
            pltpu.make_async_copy(
                rows[p], xs_hbm.at[pl.ds(0, CHUNK)], osem[p]).wait()
            _fire_half_gathers(x_hbm, idx_v, rows, gsa, gsb, i + NBUF, p)
    for p in range(NBUF):
        pltpu.make_async_copy(
            rows[p], xs_hbm.at[pl.ds(0, CHUNK)], osem[p]).wait()


def _gather_xs(x, tos):
    fn = pl.kernel(
        _gather_body,
        out_type=jax.ShapeDtypeStruct((SLOTS, D), jnp.float32),
        mesh=plsc.VectorSubcoreMesh(core_axis_name="c", subcore_axis_name="s"),
        scratch_types=(
            [pltpu.VMEM((PER_TILE,), jnp.int32)]
            + [pltpu.VMEM((CHUNK, D), jnp.float32)] * NBUF
            + [pltpu.SemaphoreType.DMA] * (3 * NBUF)
        ),
    )
    return fn(x, tos)


def _ffn_body(nblk_ref, poffb_ref, xs_ref, sg_ref, w1_hbm, b1_ref, w2_hbm,
              b2_ref, o_ref, w1s, w2s, sem1, sem2):
    e = pl.program_id(0)
    j = pl.program_id(1)

    @pl.when((j == 0) & (nblk_ref[e] > 0))
    def _():
        pltpu.make_async_copy(w1_hbm.at[e], w1s, sem1).start()
        pltpu.make_async_copy(w2_hbm.at[e], w2s, sem2).start()

    @pl.when(j < nblk_ref[e])
    def _():
        xb = xs_ref[...]

        @pl.when(j == 0)
        def _():
            pltpu.make_async_copy(w1_hbm.at[e], w1s, sem1).wait()
        hid = jax.lax.dot_general(
            xb, w1s[...], (((1,), (1,)), ((), ())),
            preferred_element_type=jnp.float32,
        ) + b1_ref[0]
        hid = jnp.maximum(hid, 0.0)

        @pl.when(j == 0)
        def _():
            pltpu.make_async_copy(w2_hbm.at[e], w2s, sem2).wait()
        o_ref[...] = (
            jax.lax.dot_general(
                hid, w2s[...], (((1,), (1,)), ((), ())),
                preferred_element_type=jnp.float32,
            ) + b2_ref[0]
        ) * sg_ref[0, 0][:, None]


def _ffn(xs, sg, nblk, poffb, w1, b1, w2, b2):
    grid_spec = pltpu.PrefetchScalarGridSpec(
        num_scalar_prefetch=2,
        grid=(E, JMAX),
        in_specs=[
            pl.BlockSpec(
                (B, D),
                lambda e, j, nblk, poffb: (jnp.minimum(poffb[e] + j, G - 1), 0),
            ),
            pl.BlockSpec(
                (1, 1, B),
                lambda e, j, nblk, poffb: (
                    jnp.where(j < nblk[e], poffb[e] + j, G), 0, 0),
            ),
            pl.BlockSpec(memory_space=pl.ANY),
            pl.BlockSpec((1, 1, H), lambda e, j, nblk, poffb: (e, 0, 0)),
            pl.BlockSpec(memory_space=pl.ANY),
            pl.BlockSpec((1, 1, D), lambda e, j, nblk, poffb: (e, 0, 0)),
        ],
        out_specs=pl.BlockSpec(
            (B, D),
            lambda e, j, nblk, poffb: (
                jnp.where(j < nblk[e], poffb[e] + j, G), 0),
        ),
        scratch_shapes=[
            pltpu.VMEM((H, D), jnp.float32),
            pltpu.VMEM((D, H), jnp.float32),
            pltpu.SemaphoreType.DMA,
            pltpu.SemaphoreType.DMA,
        ],
    )
    y = pl.pallas_call(
        _ffn_body,
        grid_spec=grid_spec,
        out_shape=jax.ShapeDtypeStruct((SLOTS + B, D), jnp.float32),
    )(nblk, poffb, xs, sg.reshape(G + 1, 1, B), w1, b1.reshape(E, 1, H),
      w2, b2.reshape(E, 1, D))
    return y


TCHUNK = 16
TOK_PER_TILE = T // NTILES


CSTEP = TOK_PER_TILE // TCHUNK


def _combine_body(yg_hbm, pos0_hbm, pos1_hbm, out_hbm, i0_v, i1_v,
                  r0a_v, r0b_v, r1a_v, r1b_v,
                  g0a, g0b, g1a, g1b, oa, ob):
    wid = lax.axis_index("s") * 2 + lax.axis_index("c")
    base = wid * TOK_PER_TILE
    pltpu.sync_copy(pos0_hbm.at[pl.ds(base, TOK_PER_TILE)], i0_v)
    pltpu.sync_copy(pos1_hbm.at[pl.ds(base, TOK_PER_TILE)], i1_v)
    r0 = (r0a_v, r0b_v)
    r1 = (r1a_v, r1b_v)
    g0 = (g0a, g0b)
    g1 = (g1a, g1b)
    osem = (oa, ob)

    for p in range(2):
        pltpu.async_copy(
            yg_hbm.at[i0_v.at[pl.ds(p * TCHUNK, TCHUNK)]], r0[p], g0[p])
        pltpu.async_copy(
            yg_hbm.at[i1_v.at[pl.ds(p * TCHUNK, TCHUNK)]], r1[p], g1[p])

    for i in range(CSTEP):
        p = i % 2
        pltpu.make_async_copy(
            yg_hbm.at[i0_v.at[pl.ds(0, TCHUNK)]], r0[p], g0[p]).wait()
        pltpu.make_async_copy(
            yg_hbm.at[i1_v.at[pl.ds(0, TCHUNK)]], r1[p], g1[p]).wait()

        def add_row(r, _, p=p):
            for q in range(D // L):
                r0[p][r, pl.ds(q * L, L)] += r1[p][r, pl.ds(q * L, L)]
            return 0
        lax.fori_loop(0, TCHUNK, add_row, 0)
        pltpu.async_copy(
            r0[p], out_hbm.at[pl.ds(base + i * TCHUNK, TCHUNK)], osem[p])
        if i + 2 < CSTEP:
            pltpu.make_async_copy(
                r0[p], out_hbm.at[pl.ds(0, TCHUNK)], osem[p]).wait()
            pltpu.async_copy(
                yg_hbm.at[i0_v.at[pl.ds((i + 2) * TCHUNK, TCHUNK)]],
                r0[p], g0[p])
            pltpu.async_copy(
                yg_hbm.at[i1_v.at[pl.ds((i + 2) * TCHUNK, TCHUNK)]],
                r1[p], g1[p])
    for p in range(2):
        pltpu.make_async_copy(
            r0[p], out_hbm.at[pl.ds(0, TCHUNK)], osem[p]).wait()


def _combine(yg, pos0, pos1):
    fn = pl.kernel(
        _combine_body,
        out_type=jax.ShapeDtypeStruct((T, D), jnp.float32),
        mesh=plsc.VectorSubcoreMesh(core_axis_name="c", subcore_axis_name="s"),
        scratch_types=[
            pltpu.VMEM((TOK_PER_TILE,), jnp.int32),
            pltpu.VMEM((TOK_PER_TILE,), jnp.int32),
            pltpu.VMEM((TCHUNK, D), jnp.float32),
            pltpu.VMEM((TCHUNK, D), jnp.float32),
            pltpu.VMEM((TCHUNK, D), jnp.float32),
            pltpu.VMEM((TCHUNK, D), jnp.float32),
            pltpu.SemaphoreType.DMA,
            pltpu.SemaphoreType.DMA,
            pltpu.SemaphoreType.DMA,
            pltpu.SemaphoreType.DMA,
            pltpu.SemaphoreType.DMA,
            pltpu.SemaphoreType.DMA,
        ],
    )
    return fn(yg, pos0, pos1)


def kernel(x, router_w, router_b, noisy_w, noisy_b, w1, b1, w2, b2):
    del noisy_w, noisy_b
    topk_idx, gates, counts = _router(x, router_w, router_b)
    pos0, pos1, tos, sg, nblk, poffb = _dispatch(
        topk_idx.reshape(N), gates.reshape(N), counts)
    xs = _gather_xs(x, tos)
    yg = _ffn(xs, sg, nblk, poffb, w1, b1, w2, b2)
    return _combine(yg, pos0, pos1)


# speedup vs baseline: 1.0326x; 1.0021x over previous
import functools

import jax
import jax.numpy as jnp
from jax import lax
from jax.experimental import pallas as pl
from jax.experimental.pallas import tpu as pltpu
from jax.experimental.pallas import tpu_sc as plsc

E = 8
K = 2
T = 4096
D = 1024
H = 4096
N = T * K
B = 256
LOGB = 8
SLOTS = N + E * B
G = SLOTS // B
JMAX = N // B + 1
L = 16
NTILES = 32


def _router_body(x_ref, rw_ref, rb_ref, idx_ref, gate_ref, cnt_ref):
    t = pl.program_id(0)
    xb = x_ref[...]
    logits = (
        jax.lax.dot_general(
            xb, rw_ref[...], (((1,), (1,)), ((), ())),
            preferred_element_type=jnp.float32,
        )
        + rb_ref[...][None, :]
    )
    m1 = jnp.max(logits, axis=-1)
    a1 = jnp.argmax(logits, axis=-1).astype(jnp.int32)
    cols = jax.lax.broadcasted_iota(jnp.int32, logits.shape, 1)
    masked = jnp.where(cols == a1[:, None], -jnp.inf, logits)
    m2 = jnp.max(masked, axis=-1)
    a2 = jnp.argmax(masked, axis=-1).astype(jnp.int32)
    e2 = jnp.exp(m2 - m1)
    denom = 1.0 + e2
    idx_ref[...] = jnp.stack([a1, a2], axis=-1)
    gate_ref[...] = jnp.stack([1.0 / denom, e2 / denom], axis=-1)

    cols16 = jax.lax.broadcasted_iota(jnp.int32, (xb.shape[0], L), 1)
    hist = jnp.sum(
        (cols16 == a1[:, None]).astype(jnp.int32)
        + (cols16 == a2[:, None]).astype(jnp.int32),
        axis=0,
    )[None, :]

    @pl.when(t == 0)
    def _():
        cnt_ref[...] = hist

    @pl.when(t > 0)
    def _():
        cnt_ref[...] += hist


def _router(x, router_w, router_b):
    return pl.pallas_call(
        _router_body,
        grid=(4,),
        in_specs=[
            pl.BlockSpec((T // 4, D), lambda t: (t, 0)),
            pl.BlockSpec((E, D), lambda t: (0, 0)),
            pl.BlockSpec((E,), lambda t: (0,)),
        ],
        out_specs=[
            pl.BlockSpec((T // 4, K), lambda t: (t, 0)),
            pl.BlockSpec((T // 4, K), lambda t: (t, 0)),
            pl.BlockSpec((1, L), lambda t: (0, 0)),
        ],
        out_shape=[
            jax.ShapeDtypeStruct((T, K), jnp.int32),
            jax.ShapeDtypeStruct((T, K), jnp.float32),
            jax.ShapeDtypeStruct((1, L), jnp.int32),
        ],
    )(x, router_w, router_b)


def _dispatch_body(ea_hbm, gf_hbm, cnt_hbm, pos0_hbm, pos1_hbm, tos_hbm,
                   sg_hbm, nblk_hbm, poffb_hbm,
                   ea_v, gf_v, pos_v, pos0_v, pos1_v, tos_v, sg_v, rp_v,
                   misc_v):
    wid = lax.axis_index("s") * 2 + lax.axis_index("c")

    @pl.when(wid == 0)
    def _():
        pltpu.sync_copy(ea_hbm, ea_v)
        pltpu.sync_copy(gf_hbm, gf_v)
        pltpu.sync_copy(cnt_hbm.at[0], misc_v.at[0])

        lanes = lax.iota(jnp.int32, L)
        counts = misc_v[0, :]
        padded = ((counts + (B - 1)) >> LOGB) << LOGB
        poff = plsc.cumsum(padded) - padded
        misc_v[1, :] = padded >> LOGB
        misc_v[2, :] = poff >> LOGB
        rp_v[...] = poff

        def zero_body(i, _):
            tos_v[pl.ds(i * L, L)] = jnp.zeros((L,), jnp.int32)
            return 0
        lax.fori_loop(0, SLOTS // L, zero_body, 0)

        def zero_sg(i, _):
            sg_v[pl.ds(i * L, L)] = jnp.zeros((L,), jnp.float32)
            return 0
        lax.fori_loop(0, (SLOTS + B) // L, zero_sg, 0)

        def chunk_body(c, _):
            v = ea_v[pl.ds(c * L, L)]
            rk = jnp.zeros((L,), jnp.int32)
            rp = rp_v[...]
            for e in range(E):
                m = v == e
                ones = jnp.where(m, 1, 0)
                cs = plsc.cumsum(ones)
                rk = jnp.where(m, cs - 1, rk)
                tot = jnp.sum(ones)
                rp = jnp.where(lanes == e, rp + tot, rp)
            base = plsc.load_gather(rp_v, [v])
            rp_v[...] = rp
            posc = base + rk
            pos_v[pl.ds(c * L, L)] = posc
            tok = (c * (L // K)) + (lanes >> 1)
            plsc.store_scatter(tos_v, [posc], tok)
            gc = gf_v[pl.ds(c * L, L)]
            plsc.store_scatter(sg_v, [posc], gc)
            return 0
        lax.fori_loop(0, N // L, chunk_body, 0)

        def split_body(c, _):
            ti = c * L + lanes
            pos0_v[pl.ds(c * L, L)] = plsc.load_gather(pos_v, [ti * 2])
            pos1_v[pl.ds(c * L, L)] = plsc.load_gather(pos_v, [ti * 2 + 1])
            return 0
        lax.fori_loop(0, T // L, split_body, 0)

        pltpu.sync_copy(pos0_v, pos0_hbm)
        pltpu.sync_copy(pos1_v, pos1_hbm)
        pltpu.sync_copy(tos_v, tos_hbm)
        pltpu.sync_copy(sg_v, sg_hbm)
        pltpu.sync_copy(misc_v.at[1], nblk_hbm)
        pltpu.sync_copy(misc_v.at[2], poffb_hbm)


def _dispatch(ea, gates_flat, counts):
    fn = pl.kernel(
        _dispatch_body,
        out_type=[
            jax.ShapeDtypeStruct((T,), jnp.int32),
            jax.ShapeDtypeStruct((T,), jnp.int32),
            jax.ShapeDtypeStruct((SLOTS,), jnp.int32),
            jax.ShapeDtypeStruct((SLOTS + B,), jnp.float32),
            jax.ShapeDtypeStruct((L,), jnp.int32),
            jax.ShapeDtypeStruct((L,), jnp.int32),
        ],
        mesh=plsc.VectorSubcoreMesh(core_axis_name="c", subcore_axis_name="s"),
        scratch_types=[
            pltpu.VMEM((N,), jnp.int32),
            pltpu.VMEM((N,), jnp.float32),
            pltpu.VMEM((N,), jnp.int32),
            pltpu.VMEM((T,), jnp.int32),
            pltpu.VMEM((T,), jnp.int32),
            pltpu.VMEM((SLOTS,), jnp.int32),
            pltpu.VMEM((SLOTS + B,), jnp.float32),
            pltpu.VMEM((L,), jnp.int32),
            pltpu.VMEM((3, L), jnp.int32),
        ],
        compiler_params=pltpu.CompilerParams(needs_layout_passes=False),
    )
    return fn(ea, gates_flat, counts)


CHUNK = 16
NBUF = 6
PER_TILE = SLOTS // NTILES
NSTEP = PER_TILE // CHUNK


def _gather_body(x_hbm, tos_hbm, xs_hbm, idx_v, *bufs_and_sems):
    rows = bufs_and_sems[:NBUF]
    gsem = bufs_and_sems[NBUF:2 * NBUF]
    osem = bufs_and_sems[2 * NBUF:3 * NBUF]
    wid = lax.axis_index("s") * 2 + lax.axis_index("c")
    base = wid * PER_TILE
    pltpu.sync_copy(tos_hbm.at[pl.ds(base, PER_TILE)], idx_v)

    def fire(i, p):
        idxv = idx_v[pl.ds(i * CHUNK, CHUNK)]
        pltpu.async_copy(x_hbm.at[idxv], rows[p], gsem[p])

    for p in range(NBUF):
        fire(p, p)

    for i in range(NSTEP):
        p = i % NBUF
        pltpu.make_async_copy(
            x_hbm.at[idx_v.at[pl.ds(0, CHUNK)]], rows[p], gsem[p]).wait()
        pltpu.async_copy(
            rows[p], xs_hbm.at[pl.ds(base + i * CHUNK, CHUNK)], osem[p])
        if i + NBUF < NSTEP:
            pltpu.make_async_copy(
                rows[p], xs_hbm.at[pl.ds(0, CHUNK)], osem[p]).wait()
            fire(i + NBUF, p)
    for p in range(NBUF):
        pltpu.make_async_copy(
            rows[p], xs_hbm.at[pl.ds(0, CHUNK)], osem[p]).wait()


def _gather_xs(x, tos):
    fn = pl.kernel(
        _gather_body,
        out_type=jax.ShapeDtypeStruct((SLOTS, D), jnp.float32),
        mesh=plsc.VectorSubcoreMesh(core_axis_name="c", subcore_axis_name="s"),
        scratch_types=(
            [pltpu.VMEM((PER_TILE,), jnp.int32)]
            + [pltpu.VMEM((CHUNK, D), jnp.float32)] * NBUF
            + [pltpu.SemaphoreType.DMA] * (2 * NBUF)
        ),
    )
    return fn(x, tos)


def _ffn_body(nblk_ref, poffb_ref, xs_ref, sg_ref, w1_hbm, b1_ref, w2_hbm,
              b2_ref, o_ref, w1s, w2s, sem1, sem2):
    e = pl.program_id(0)
    j = pl.program_id(1)

    @pl.when((j == 0) & (nblk_ref[e] > 0))
    def _():
        pltpu.make_async_copy(w1_hbm.at[e], w1s, sem1).start()
        pltpu.make_async_copy(w2_hbm.at[e], w2s, sem2).start()

    @pl.when(j < nblk_ref[e])
    def _():
        xb = xs_ref[...]

        @pl.when(j == 0)
        def _():
            pltpu.make_async_copy(w1_hbm.at[e], w1s, sem1).wait()
        hid = jax.lax.dot_general(
            xb, w1s[...], (((1,), (1,)), ((), ())),
            preferred_element_type=jnp.float32,
        ) + b1_ref[0]
        hid = jnp.maximum(hid, 0.0)

        @pl.when(j == 0)
        def _():
            pltpu.make_async_copy(w2_hbm.at[e], w2s, sem2).wait()
        o_ref[...] = (
            jax.lax.dot_general(
                hid, w2s[...], (((1,), (1,)), ((), ())),
                preferred_element_type=jnp.float32,
            ) + b2_ref[0]
        ) * sg_ref[0, 0][:, None]


def _ffn(xs, sg, nblk, poffb, w1, b1, w2, b2):
    grid_spec = pltpu.PrefetchScalarGridSpec(
        num_scalar_prefetch=2,
        grid=(E, JMAX),
        in_specs=[
            pl.BlockSpec(
                (B, D),
                lambda e, j, nblk, poffb: (jnp.minimum(poffb[e] + j, G - 1), 0),
            ),
            pl.BlockSpec(
                (1, 1, B),
                lambda e, j, nblk, poffb: (
                    jnp.where(j < nblk[e], poffb[e] + j, G), 0, 0),
            ),
            pl.BlockSpec(memory_space=pl.ANY),
            pl.BlockSpec((1, 1, H), lambda e, j, nblk, poffb: (e, 0, 0)),
            pl.BlockSpec(memory_space=pl.ANY),
            pl.BlockSpec((1, 1, D), lambda e, j, nblk, poffb: (e, 0, 0)),
        ],
        out_specs=pl.BlockSpec(
            (B, D),
            lambda e, j, nblk, poffb: (
                jnp.where(j < nblk[e], poffb[e] + j, G), 0),
        ),
        scratch_shapes=[
            pltpu.VMEM((H, D), jnp.float32),
            pltpu.VMEM((D, H), jnp.float32),
            pltpu.SemaphoreType.DMA,
            pltpu.SemaphoreType.DMA,
        ],
    )
    y = pl.pallas_call(
        _ffn_body,
        grid_spec=grid_spec,
        out_shape=jax.ShapeDtypeStruct((SLOTS + B, D), jnp.float32),
    )(nblk, poffb, xs, sg.reshape(G + 1, 1, B), w1, b1.reshape(E, 1, H),
      w2, b2.reshape(E, 1, D))
    return y


TCHUNK = 16
TOK_PER_TILE = T // NTILES


CSTEP = TOK_PER_TILE // TCHUNK


def _combine_body(yg_hbm, pos0_hbm, pos1_hbm, out_hbm, i0_v, i1_v,
                  r0a_v, r0b_v, r1a_v, r1b_v,
                  g0a, g0b, g1a, g1b, oa, ob):
    wid = lax.axis_index("s") * 2 + lax.axis_index("c")
    base = wid * TOK_PER_TILE
    pltpu.sync_copy(pos0_hbm.at[pl.ds(base, TOK_PER_TILE)], i0_v)
    pltpu.sync_copy(pos1_hbm.at[pl.ds(base, TOK_PER_TILE)], i1_v)
    r0 = (r0a_v, r0b_v)
    r1 = (r1a_v, r1b_v)
    g0 = (g0a, g0b)
    g1 = (g1a, g1b)
    osem = (oa, ob)

    for p in range(2):
        pltpu.async_copy(
            yg_hbm.at[i0_v.at[pl.ds(p * TCHUNK, TCHUNK)]], r0[p], g0[p])
        pltpu.async_copy(
            yg_hbm.at[i1_v.at[pl.ds(p * TCHUNK, TCHUNK)]], r1[p], g1[p])

    for i in range(CSTEP):
        p = i % 2
        pltpu.make_async_copy(
            yg_hbm.at[i0_v.at[pl.ds(0, TCHUNK)]], r0[p], g0[p]).wait()
        pltpu.make_async_copy(
            yg_hbm.at[i1_v.at[pl.ds(0, TCHUNK)]], r1[p], g1[p]).wait()

        def add_row(r, _, p=p):
            for q in range(D // L):
                r0[p][r, pl.ds(q * L, L)] += r1[p][r, pl.ds(q * L, L)]
            return 0
        lax.fori_loop(0, TCHUNK, add_row, 0)
        pltpu.async_copy(
            r0[p], out_hbm.at[pl.ds(base + i * TCHUNK, TCHUNK)], osem[p])
        if i + 2 < CSTEP:
            pltpu.make_async_copy(
                r0[p], out_hbm.at[pl.ds(0, TCHUNK)], osem[p]).wait()
            pltpu.async_copy(
                yg_hbm.at[i0_v.at[pl.ds((i + 2) * TCHUNK, TCHUNK)]],
                r0[p], g0[p])
            pltpu.async_copy(
                yg_hbm.at[i1_v.at[pl.ds((i + 2) * TCHUNK, TCHUNK)]],
                r1[p], g1[p])
    for p in range(2):
        pltpu.make_async_copy(
            r0[p], out_hbm.at[pl.ds(0, TCHUNK)], osem[p]).wait()


def _combine(yg, pos0, pos1):
    fn = pl.kernel(
        _combine_body,
        out_type=jax.ShapeDtypeStruct((T, D), jnp.float32),
        mesh=plsc.VectorSubcoreMesh(core_axis_name="c", subcore_axis_name="s"),
        scratch_types=[
            pltpu.VMEM((TOK_PER_TILE,), jnp.int32),
            pltpu.VMEM((TOK_PER_TILE,), jnp.int32),
            pltpu.VMEM((TCHUNK, D), jnp.float32),
            pltpu.VMEM((TCHUNK, D), jnp.float32),
            pltpu.VMEM((TCHUNK, D), jnp.float32),
            pltpu.VMEM((TCHUNK, D), jnp.float32),
            pltpu.SemaphoreType.DMA,
            pltpu.SemaphoreType.DMA,
            pltpu.SemaphoreType.DMA,
            pltpu.SemaphoreType.DMA,
            pltpu.SemaphoreType.DMA,
            pltpu.SemaphoreType.DMA,
        ],
    )
    return fn(yg, pos0, pos1)


def kernel(x, router_w, router_b, noisy_w, noisy_b, w1, b1, w2, b2):
    del noisy_w, noisy_b
    topk_idx, gates, counts = _router(x, router_w, router_b)
    pos0, pos1, tos, sg, nblk, poffb = _dispatch(
        topk_idx.reshape(N), gates.reshape(N), counts)
    xs = _gather_xs(x, tos)
    yg = _ffn(xs, sg, nblk, poffb, w1, b1, w2, b2)
    return _combine(yg, pos0, pos1)
